# baseline (speedup 1.0000x reference)
import functools

import jax
import jax.numpy as jnp
from jax import lax
from jax.experimental import pallas as pl
from jax.experimental.pallas import tpu as pltpu
from jax.experimental.pallas import tpu_sc as plsc

_N = 10000
_N_PAD = 10240
_E = 320000
_IN_C = 128
_HID = 256
_OUT_C = 64

_K = 112
_NCHUNK = 90
_NCHUNK0 = 152
_NCHUNK1 = 28
_LANES = 16
_NROW_SLOT = 3
_NIDX_SLOT = 4


@functools.lru_cache(maxsize=None)
def _make_spmm(n_nodes, d, tc_tiling):
    info = plsc.get_sparse_core_info()
    nc, ns = int(info.num_cores), int(info.num_subcores)
    rows_per_tile = _N_PAD // ns
    ngroup = _K // _LANES

    mesh = plsc.VectorSubcoreMesh(core_axis_name="c", subcore_axis_name="s")

    @functools.partial(
        pl.kernel,
        mesh=mesh,
        compiler_params=pltpu.CompilerParams(use_tc_tiling_on_sc=tc_tiling),
        out_type=jax.ShapeDtypeStruct((nc, _N_PAD, d), jnp.float32),
        scratch_types=[
            pltpu.VMEM((_NIDX_SLOT, 2, _K), jnp.int32),
            pltpu.VMEM((_NIDX_SLOT, _K), jnp.float32),
            pltpu.VMEM_SHARED((_N_PAD, d), jnp.float32),
            pltpu.VMEM((_NROW_SLOT, _K, d), jnp.float32),
            pltpu.SemaphoreType.DMA((_NIDX_SLOT,)),
            pltpu.SemaphoreType.DMA((_NIDX_SLOT,)),
            pltpu.SemaphoreType.DMA((_NIDX_SLOT,)),
            pltpu.SemaphoreType.DMA((_NROW_SLOT,)),
            pltpu.SemaphoreType.DMA((_NROW_SLOT,)),
        ],
    )
    def spmm(x_hbm, src_hbm, dst_hbm, ew_hbm, out_hbm, idx_v, ew_v, agg_sh,
             rows_v, isem, dsem, esem, gsem, ssem):
        cid = lax.axis_index("c")
        sid = lax.axis_index("s")
        nchunk = jnp.where(cid == 0, _NCHUNK0, _NCHUNK1)
        qbase = jnp.where(cid == 0, sid * _NCHUNK0,
                          ns * _NCHUNK0 + sid * _NCHUNK1)

        base = sid * rows_per_tile

        def issue_idx(m):
            s = m % _NIDX_SLOT
            e0 = (qbase + m) * _K
            pltpu.async_copy(src_hbm.at[pl.ds(e0, _K)], idx_v.at[s, 0],
                             isem.at[s])
            pltpu.async_copy(dst_hbm.at[pl.ds(e0, _K)], idx_v.at[s, 1],
                             dsem.at[s])
            pltpu.async_copy(ew_hbm.at[pl.ds(e0, _K)], ew_v.at[s], esem.at[s])

        def wait_idx(m):
            s = m % _NIDX_SLOT
            e0 = (qbase + m) * _K
            pltpu.make_async_copy(src_hbm.at[pl.ds(e0, _K)], idx_v.at[s, 0],
                                  isem.at[s]).wait()
            pltpu.make_async_copy(dst_hbm.at[pl.ds(e0, _K)], idx_v.at[s, 1],
                                  dsem.at[s]).wait()
            pltpu.make_async_copy(ew_hbm.at[pl.ds(e0, _K)], ew_v.at[s],
                                  esem.at[s]).wait()

        def issue_gather(m):
            s, r = m % _NIDX_SLOT, m % _NROW_SLOT
            pltpu.async_copy(x_hbm.at[idx_v.at[s, 0]], rows_v.at[r],
                             gsem.at[r])

        def wait_gather(m):
            s, r = m % _NIDX_SLOT, m % _NROW_SLOT
            pltpu.make_async_copy(x_hbm.at[idx_v.at[s, 0]], rows_v.at[r],
                                  gsem.at[r]).wait()

        def issue_scatter(m):
            s, r = m % _NIDX_SLOT, m % _NROW_SLOT
            pltpu.async_copy(rows_v.at[r], agg_sh.at[idx_v.at[s, 1]],
                             ssem.at[r], add=True)

        def wait_scatter(m):
            s, r = m % _NIDX_SLOT, m % _NROW_SLOT
            pltpu.make_async_copy(rows_v.at[r], agg_sh.at[idx_v.at[s, 1]],
                                  ssem.at[r]).wait()

        def scale(m):
            s, r = m % _NIDX_SLOT, m % _NROW_SLOT

            @plsc.parallel_loop(0, ngroup, unroll=7)
            def grp(g):
                eww = ew_v[s, pl.ds(g * _LANES, _LANES)]
                for l in range(_LANES):
                    w = eww[l]
                    for c in range(d // _LANES):
                        sl = pl.ds(c * _LANES, _LANES)
                        rows_v[r, g * _LANES + l, sl] = \
                            rows_v[r, g * _LANES + l, sl] * w

        issue_idx(0)
        issue_idx(1)
        issue_idx(2)

        zeros16 = jnp.zeros((_LANES,), jnp.float32)
        zslot = _NROW_SLOT - 1

        def zrow(i, carry):
            for g in range(d // _LANES):
                rows_v[zslot, i, pl.ds(g * _LANES, _LANES)] = zeros16
            return carry

        lax.fori_loop(0, _K, zrow, 0)

        wait_idx(0)
        issue_gather(0)
        wait_idx(1)
        issue_gather(1)

        zcopies = []
        off = 0
        while off < rows_per_tile:
            nrow = min(_K, rows_per_tile - off)
            zcopies.append((off, nrow))
            off += nrow
        for off, nrow in zcopies:
            pltpu.async_copy(rows_v.at[zslot, pl.ds(0, nrow)],
                             agg_sh.at[pl.ds(base + off, nrow)],
                             ssem.at[zslot])
        for off, nrow in zcopies:
            pltpu.make_async_copy(rows_v.at[zslot, pl.ds(0, nrow)],
                                  agg_sh.at[pl.ds(base + off, nrow)],
                                  ssem.at[zslot]).wait()
        plsc.subcore_barrier()

        def body(j, first, last_idx, last_gather):
            wait_gather(j)
            scale(j)
            issue_scatter(j)
            if not first:
                wait_scatter(j - 1)
            if not last_gather:
                wait_idx(j + 2)
                issue_gather(j + 2)
            if not last_idx:
                issue_idx(j + 3)
            return 0

        body(0, True, False, False)
        lax.fori_loop(1, nchunk - 3,
                      lambda j, c: body(j, False, False, False), 0)
        body(nchunk - 3, False, True, False)
        body(nchunk - 2, False, True, True)
        body(nchunk - 1, False, True, True)
        wait_scatter(nchunk - 1)
        plsc.subcore_barrier()

        pltpu.sync_copy(agg_sh.at[pl.ds(base, rows_per_tile)],
                        out_hbm.at[cid, pl.ds(base, rows_per_tile)])

    return spmm


def _tc_mid(p0, p1, W_in, W_out, b_in):
    bm = 2000

    def body(p0_ref, p1_ref, wi_ref, wo_ref, bi_ref, o_ref):
        w12 = jnp.dot(wi_ref[...], wo_ref[...], preferred_element_type=jnp.float32)
        b12 = jnp.dot(bi_ref[...], wo_ref[...], preferred_element_type=jnp.float32)
        s = p0_ref[...] + p1_ref[...]
        o_ref[...] = jnp.dot(s, w12, preferred_element_type=jnp.float32) + b12

    return pl.pallas_call(
        body,
        grid=(_N // bm,),
        in_specs=[
            pl.BlockSpec((bm, _IN_C), lambda i: (i, 0)),
            pl.BlockSpec((bm, _IN_C), lambda i: (i, 0)),
            pl.BlockSpec((_IN_C, _HID), lambda i: (0, 0)),
            pl.BlockSpec((_HID, _OUT_C), lambda i: (0, 0)),
            pl.BlockSpec((1, _HID), lambda i: (0, 0)),
        ],
        out_specs=pl.BlockSpec((bm, _OUT_C), lambda i: (i, 0)),
        out_shape=jax.ShapeDtypeStruct((_N, _OUT_C), jnp.float32),
    )(p0, p1, W_in, W_out, b_in.reshape(1, _HID))


def _tc_out(q0, q1, b_out):
    bm = 2000

    def body(q0_ref, q1_ref, b_ref, o_ref):
        z = (q0_ref[...] + q1_ref[...]) + b_ref[...]
        m = jnp.max(z, axis=1, keepdims=True)
        e = jnp.exp(z - m)
        o_ref[...] = (z - m) - jnp.log(jnp.sum(e, axis=1, keepdims=True))

    return pl.pallas_call(
        body,
        grid=(_N // bm,),
        in_specs=[
            pl.BlockSpec((bm, _OUT_C), lambda i: (i, 0)),
            pl.BlockSpec((bm, _OUT_C), lambda i: (i, 0)),
            pl.BlockSpec((1, _OUT_C), lambda i: (0, 0)),
        ],
        out_specs=pl.BlockSpec((bm, _OUT_C), lambda i: (i, 0)),
        out_shape=jax.ShapeDtypeStruct((_N, _OUT_C), jnp.float32),
    )(q0, q1, b_out.reshape(1, _OUT_C))


def kernel(x, adj, edge_weight, W_in, b_in, W_out, b_out):
    nw = 32
    ep = nw * _NCHUNK * _K
    pad = ep - _E
    src = jnp.concatenate([adj[0], jnp.zeros((pad,), jnp.int32)])
    dst = jnp.concatenate([adj[1], jnp.zeros((pad,), jnp.int32)])
    ew = jnp.concatenate([edge_weight, jnp.zeros((pad,), jnp.float32)])

    p1 = _make_spmm(_N, _IN_C, True)(x, src, dst, ew)
    h2 = _tc_mid(p1[0], p1[1], W_in, W_out, b_in)
    p2 = _make_spmm(_N, _OUT_C, False)(h2, src, dst, ew)
    return _tc_out(p2[0], p2[1], b_out)


# speedup vs baseline: 1.0355x; 1.0355x over previous
import functools

import jax
import jax.numpy as jnp
from jax import lax
from jax.experimental import pallas as pl
from jax.experimental.pallas import tpu as pltpu
from jax.experimental.pallas import tpu_sc as plsc

_N = 10000
_N_PAD = 10240
_E = 320000
_IN_C = 128
_HID = 256
_OUT_C = 64

_K = 112
_NCHUNK = 90
_NCHUNK0 = 144
_NCHUNK1 = 36
_LANES = 16
_NROW_SLOT = 3
_NIDX_SLOT = 4


@functools.lru_cache(maxsize=None)
def _make_spmm(n_nodes, d, tc_tiling):
    info = plsc.get_sparse_core_info()
    nc, ns = int(info.num_cores), int(info.num_subcores)
    rows_per_tile = _N_PAD // ns
    ngroup = _K // _LANES

    mesh = plsc.VectorSubcoreMesh(core_axis_name="c", subcore_axis_name="s")

    @functools.partial(
        pl.kernel,
        mesh=mesh,
        compiler_params=pltpu.CompilerParams(use_tc_tiling_on_sc=tc_tiling),
        out_type=jax.ShapeDtypeStruct((nc, _N_PAD, d), jnp.float32),
        scratch_types=[
            pltpu.VMEM((_NIDX_SLOT, 2, _K), jnp.int32),
            pltpu.VMEM((_NIDX_SLOT, _K), jnp.float32),
            pltpu.VMEM_SHARED((_N_PAD, d), jnp.float32),
            pltpu.VMEM((_NROW_SLOT, _K, d), jnp.float32),
            pltpu.SemaphoreType.DMA((_NIDX_SLOT,)),
            pltpu.SemaphoreType.DMA((_NIDX_SLOT,)),
            pltpu.SemaphoreType.DMA((_NIDX_SLOT,)),
            pltpu.SemaphoreType.DMA((_NROW_SLOT,)),
            pltpu.SemaphoreType.DMA((_NROW_SLOT,)),
        ],
    )
    def spmm(x_hbm, src_hbm, dst_hbm, ew_hbm, out_hbm, idx_v, ew_v, agg_sh,
             rows_v, isem, dsem, esem, gsem, ssem):
        cid = lax.axis_index("c")
        sid = lax.axis_index("s")
        nchunk = jnp.where(cid == 0, _NCHUNK0, _NCHUNK1)
        qbase = jnp.where(cid == 0, sid * _NCHUNK0,
                          ns * _NCHUNK0 + sid * _NCHUNK1)

        base = sid * rows_per_tile

        def issue_idx(m):
            s = m % _NIDX_SLOT
            e0 = (qbase + m) * _K
            pltpu.async_copy(src_hbm.at[pl.ds(e0, _K)], idx_v.at[s, 0],
                             isem.at[s])
            pltpu.async_copy(dst_hbm.at[pl.ds(e0, _K)], idx_v.at[s, 1],
                             dsem.at[s])
            pltpu.async_copy(ew_hbm.at[pl.ds(e0, _K)], ew_v.at[s], esem.at[s])

        def wait_idx(m):
            s = m % _NIDX_SLOT
            e0 = (qbase + m) * _K
            pltpu.make_async_copy(src_hbm.at[pl.ds(e0, _K)], idx_v.at[s, 0],
                                  isem.at[s]).wait()
            pltpu.make_async_copy(dst_hbm.at[pl.ds(e0, _K)], idx_v.at[s, 1],
                                  dsem.at[s]).wait()
            pltpu.make_async_copy(ew_hbm.at[pl.ds(e0, _K)], ew_v.at[s],
                                  esem.at[s]).wait()

        def issue_gather(m):
            s, r = m % _NIDX_SLOT, m % _NROW_SLOT
            pltpu.async_copy(x_hbm.at[idx_v.at[s, 0]], rows_v.at[r],
                             gsem.at[r])

        def wait_gather(m):
            s, r = m % _NIDX_SLOT, m % _NROW_SLOT
            pltpu.make_async_copy(x_hbm.at[idx_v.at[s, 0]], rows_v.at[r],
                                  gsem.at[r]).wait()

        def issue_scatter(m):
            s, r = m % _NIDX_SLOT, m % _NROW_SLOT
            pltpu.async_copy(rows_v.at[r], agg_sh.at[idx_v.at[s, 1]],
                             ssem.at[r], add=True)

        def wait_scatter(m):
            s, r = m % _NIDX_SLOT, m % _NROW_SLOT
            pltpu.make_async_copy(rows_v.at[r], agg_sh.at[idx_v.at[s, 1]],
                                  ssem.at[r]).wait()

        def scale(m):
            s, r = m % _NIDX_SLOT, m % _NROW_SLOT

            @plsc.parallel_loop(0, ngroup, unroll=7)
            def grp(g):
                eww = ew_v[s, pl.ds(g * _LANES, _LANES)]
                for l in range(_LANES):
                    w = eww[l]
                    for c in range(d // _LANES):
                        sl = pl.ds(c * _LANES, _LANES)
                        rows_v[r, g * _LANES + l, sl] = \
                            rows_v[r, g * _LANES + l, sl] * w

        issue_idx(0)
        issue_idx(1)
        issue_idx(2)

        zeros16 = jnp.zeros((_LANES,), jnp.float32)
        zslot = _NROW_SLOT - 1

        def zrow(i, carry):
            for g in range(d // _LANES):
                rows_v[zslot, i, pl.ds(g * _LANES, _LANES)] = zeros16
            return carry

        lax.fori_loop(0, _K, zrow, 0)

        wait_idx(0)
        issue_gather(0)
        wait_idx(1)
        issue_gather(1)

        zcopies = []
        off = 0
        while off < rows_per_tile:
            nrow = min(_K, rows_per_tile - off)
            zcopies.append((off, nrow))
            off += nrow
        for off, nrow in zcopies:
            pltpu.async_copy(rows_v.at[zslot, pl.ds(0, nrow)],
                             agg_sh.at[pl.ds(base + off, nrow)],
                             ssem.at[zslot])
        for off, nrow in zcopies:
            pltpu.make_async_copy(rows_v.at[zslot, pl.ds(0, nrow)],
                                  agg_sh.at[pl.ds(base + off, nrow)],
                                  ssem.at[zslot]).wait()
        plsc.subcore_barrier()

        def body(j, first, last_idx, last_gather):
            wait_gather(j)
            scale(j)
            issue_scatter(j)
            if not first:
                wait_scatter(j - 1)
            if not last_gather:
                wait_idx(j + 2)
                issue_gather(j + 2)
            if not last_idx:
                issue_idx(j + 3)
            return 0

        body(0, True, False, False)
        lax.fori_loop(1, nchunk - 3,
                      lambda j, c: body(j, False, False, False), 0)
        body(nchunk - 3, False, True, False)
        body(nchunk - 2, False, True, True)
        body(nchunk - 1, False, True, True)
        wait_scatter(nchunk - 1)
        plsc.subcore_barrier()

        pltpu.sync_copy(agg_sh.at[pl.ds(base, rows_per_tile)],
                        out_hbm.at[cid, pl.ds(base, rows_per_tile)])

    return spmm


def _tc_mid(p0, p1, W_in, W_out, b_in):
    bm = 2000

    def body(p0_ref, p1_ref, wi_ref, wo_ref, bi_ref, o_ref):
        w12 = jnp.dot(wi_ref[...], wo_ref[...], preferred_element_type=jnp.float32)
        b12 = jnp.dot(bi_ref[...], wo_ref[...], preferred_element_type=jnp.float32)
        s = p0_ref[...] + p1_ref[...]
        o_ref[...] = jnp.dot(s, w12, preferred_element_type=jnp.float32) + b12

    return pl.pallas_call(
        body,
        grid=(_N // bm,),
        in_specs=[
            pl.BlockSpec((bm, _IN_C), lambda i: (i, 0)),
            pl.BlockSpec((bm, _IN_C), lambda i: (i, 0)),
            pl.BlockSpec((_IN_C, _HID), lambda i: (0, 0)),
            pl.BlockSpec((_HID, _OUT_C), lambda i: (0, 0)),
            pl.BlockSpec((1, _HID), lambda i: (0, 0)),
        ],
        out_specs=pl.BlockSpec((bm, _OUT_C), lambda i: (i, 0)),
        out_shape=jax.ShapeDtypeStruct((_N, _OUT_C), jnp.float32),
    )(p0, p1, W_in, W_out, b_in.reshape(1, _HID))


def _tc_out(q0, q1, b_out):
    bm = 2000

    def body(q0_ref, q1_ref, b_ref, o_ref):
        z = (q0_ref[...] + q1_ref[...]) + b_ref[...]
        m = jnp.max(z, axis=1, keepdims=True)
        e = jnp.exp(z - m)
        o_ref[...] = (z - m) - jnp.log(jnp.sum(e, axis=1, keepdims=True))

    return pl.pallas_call(
        body,
        grid=(_N // bm,),
        in_specs=[
            pl.BlockSpec((bm, _OUT_C), lambda i: (i, 0)),
            pl.BlockSpec((bm, _OUT_C), lambda i: (i, 0)),
            pl.BlockSpec((1, _OUT_C), lambda i: (0, 0)),
        ],
        out_specs=pl.BlockSpec((bm, _OUT_C), lambda i: (i, 0)),
        out_shape=jax.ShapeDtypeStruct((_N, _OUT_C), jnp.float32),
    )(q0, q1, b_out.reshape(1, _OUT_C))


def kernel(x, adj, edge_weight, W_in, b_in, W_out, b_out):
    nw = 32
    ep = nw * _NCHUNK * _K
    pad = ep - _E
    src = jnp.concatenate([adj[0], jnp.zeros((pad,), jnp.int32)])
    dst = jnp.concatenate([adj[1], jnp.zeros((pad,), jnp.int32)])
    ew = jnp.concatenate([edge_weight, jnp.zeros((pad,), jnp.float32)])

    p1 = _make_spmm(_N, _IN_C, True)(x, src, dst, ew)
    h2 = _tc_mid(p1[0], p1[1], W_in, W_out, b_in)
    p2 = _make_spmm(_N, _OUT_C, False)(h2, src, dst, ew)
    return _tc_out(p2[0], p2[1], b_out)


# speedup vs baseline: 1.0497x; 1.0138x over previous
import functools

import jax
import jax.numpy as jnp
from jax import lax
from jax.experimental import pallas as pl
from jax.experimental.pallas import tpu as pltpu
from jax.experimental.pallas import tpu_sc as plsc

_N = 10000
_N_PAD = 10240
_E = 320000
_IN_C = 128
_HID = 256
_OUT_C = 64

_K = 112
_NCHUNK = 90
_NCHUNK0 = 136
_NCHUNK1 = 44
_LANES = 16
_NROW_SLOT = 3
_NIDX_SLOT = 4


@functools.lru_cache(maxsize=None)
def _make_spmm(n_nodes, d, tc_tiling):
    info = plsc.get_sparse_core_info()
    nc, ns = int(info.num_cores), int(info.num_subcores)
    rows_per_tile = _N_PAD // ns
    ngroup = _K // _LANES

    mesh = plsc.VectorSubcoreMesh(core_axis_name="c", subcore_axis_name="s")

    @functools.partial(
        pl.kernel,
        mesh=mesh,
        compiler_params=pltpu.CompilerParams(use_tc_tiling_on_sc=tc_tiling),
        out_type=jax.ShapeDtypeStruct((nc, _N_PAD, d), jnp.float32),
        scratch_types=[
            pltpu.VMEM((_NIDX_SLOT, 2, _K), jnp.int32),
            pltpu.VMEM((_NIDX_SLOT, _K), jnp.float32),
            pltpu.VMEM_SHARED((_N_PAD, d), jnp.float32),
            pltpu.VMEM((_NROW_SLOT, _K, d), jnp.float32),
            pltpu.SemaphoreType.DMA((_NIDX_SLOT,)),
            pltpu.SemaphoreType.DMA((_NIDX_SLOT,)),
            pltpu.SemaphoreType.DMA((_NIDX_SLOT,)),
            pltpu.SemaphoreType.DMA((_NROW_SLOT,)),
            pltpu.SemaphoreType.DMA((_NROW_SLOT,)),
        ],
    )
    def spmm(x_hbm, src_hbm, dst_hbm, ew_hbm, out_hbm, idx_v, ew_v, agg_sh,
             rows_v, isem, dsem, esem, gsem, ssem):
        cid = lax.axis_index("c")
        sid = lax.axis_index("s")
        nchunk = jnp.where(cid == 0, _NCHUNK0, _NCHUNK1)
        qbase = jnp.where(cid == 0, sid * _NCHUNK0,
                          ns * _NCHUNK0 + sid * _NCHUNK1)

        base = sid * rows_per_tile

        def issue_idx(m):
            s = m % _NIDX_SLOT
            e0 = (qbase + m) * _K
            pltpu.async_copy(src_hbm.at[pl.ds(e0, _K)], idx_v.at[s, 0],
                             isem.at[s])
            pltpu.async_copy(dst_hbm.at[pl.ds(e0, _K)], idx_v.at[s, 1],
                             dsem.at[s])
            pltpu.async_copy(ew_hbm.at[pl.ds(e0, _K)], ew_v.at[s], esem.at[s])

        def wait_idx(m):
            s = m % _NIDX_SLOT
            e0 = (qbase + m) * _K
            pltpu.make_async_copy(src_hbm.at[pl.ds(e0, _K)], idx_v.at[s, 0],
                                  isem.at[s]).wait()
            pltpu.make_async_copy(dst_hbm.at[pl.ds(e0, _K)], idx_v.at[s, 1],
                                  dsem.at[s]).wait()
            pltpu.make_async_copy(ew_hbm.at[pl.ds(e0, _K)], ew_v.at[s],
                                  esem.at[s]).wait()

        def issue_gather(m):
            s, r = m % _NIDX_SLOT, m % _NROW_SLOT
            pltpu.async_copy(x_hbm.at[idx_v.at[s, 0]], rows_v.at[r],
                             gsem.at[r])

        def wait_gather(m):
            s, r = m % _NIDX_SLOT, m % _NROW_SLOT
            pltpu.make_async_copy(x_hbm.at[idx_v.at[s, 0]], rows_v.at[r],
                                  gsem.at[r]).wait()

        def issue_scatter(m):
            s, r = m % _NIDX_SLOT, m % _NROW_SLOT
            pltpu.async_copy(rows_v.at[r], agg_sh.at[idx_v.at[s, 1]],
                             ssem.at[r], add=True)

        def wait_scatter(m):
            s, r = m % _NIDX_SLOT, m % _NROW_SLOT
            pltpu.make_async_copy(rows_v.at[r], agg_sh.at[idx_v.at[s, 1]],
                                  ssem.at[r]).wait()

        def scale(m):
            s, r = m % _NIDX_SLOT, m % _NROW_SLOT

            @plsc.parallel_loop(0, ngroup, unroll=7)
            def grp(g):
                eww = ew_v[s, pl.ds(g * _LANES, _LANES)]
                for l in range(_LANES):
                    w = eww[l]
                    for c in range(d // _LANES):
                        sl = pl.ds(c * _LANES, _LANES)
                        rows_v[r, g * _LANES + l, sl] = \
                            rows_v[r, g * _LANES + l, sl] * w

        issue_idx(0)
        issue_idx(1)
        issue_idx(2)

        zeros16 = jnp.zeros((_LANES,), jnp.float32)
        zslot = _NROW_SLOT - 1

        def zrow(i, carry):
            for g in range(d // _LANES):
                rows_v[zslot, i, pl.ds(g * _LANES, _LANES)] = zeros16
            return carry

        lax.fori_loop(0, _K, zrow, 0)

        wait_idx(0)
        issue_gather(0)
        wait_idx(1)
        issue_gather(1)

        zcopies = []
        off = 0
        while off < rows_per_tile:
            nrow = min(_K, rows_per_tile - off)
            zcopies.append((off, nrow))
            off += nrow
        for off, nrow in zcopies:
            pltpu.async_copy(rows_v.at[zslot, pl.ds(0, nrow)],
                             agg_sh.at[pl.ds(base + off, nrow)],
                             ssem.at[zslot])
        for off, nrow in zcopies:
            pltpu.make_async_copy(rows_v.at[zslot, pl.ds(0, nrow)],
                                  agg_sh.at[pl.ds(base + off, nrow)],
                                  ssem.at[zslot]).wait()
        plsc.subcore_barrier()

        def body(j, first, last_idx, last_gather):
            wait_gather(j)
            scale(j)
            issue_scatter(j)
            if not first:
                wait_scatter(j - 1)
            if not last_gather:
                wait_idx(j + 2)
                issue_gather(j + 2)
            if not last_idx:
                issue_idx(j + 3)
            return 0

        body(0, True, False, False)
        lax.fori_loop(1, nchunk - 3,
                      lambda j, c: body(j, False, False, False), 0)
        body(nchunk - 3, False, True, False)
        body(nchunk - 2, False, True, True)
        body(nchunk - 1, False, True, True)
        wait_scatter(nchunk - 1)
        plsc.subcore_barrier()

        pltpu.sync_copy(agg_sh.at[pl.ds(base, rows_per_tile)],
                        out_hbm.at[cid, pl.ds(base, rows_per_tile)])

    return spmm


def _tc_mid(p0, p1, W_in, W_out, b_in):
    bm = 2000

    def body(p0_ref, p1_ref, wi_ref, wo_ref, bi_ref, o_ref):
        w12 = jnp.dot(wi_ref[...], wo_ref[...], preferred_element_type=jnp.float32)
        b12 = jnp.dot(bi_ref[...], wo_ref[...], preferred_element_type=jnp.float32)
        s = p0_ref[...] + p1_ref[...]
        o_ref[...] = jnp.dot(s, w12, preferred_element_type=jnp.float32) + b12

    return pl.pallas_call(
        body,
        grid=(_N // bm,),
        in_specs=[
            pl.BlockSpec((bm, _IN_C), lambda i: (i, 0)),
            pl.BlockSpec((bm, _IN_C), lambda i: (i, 0)),
            pl.BlockSpec((_IN_C, _HID), lambda i: (0, 0)),
            pl.BlockSpec((_HID, _OUT_C), lambda i: (0, 0)),
            pl.BlockSpec((1, _HID), lambda i: (0, 0)),
        ],
        out_specs=pl.BlockSpec((bm, _OUT_C), lambda i: (i, 0)),
        out_shape=jax.ShapeDtypeStruct((_N, _OUT_C), jnp.float32),
    )(p0, p1, W_in, W_out, b_in.reshape(1, _HID))


def _tc_out(q0, q1, b_out):
    bm = 2000

    def body(q0_ref, q1_ref, b_ref, o_ref):
        z = (q0_ref[...] + q1_ref[...]) + b_ref[...]
        m = jnp.max(z, axis=1, keepdims=True)
        e = jnp.exp(z - m)
        o_ref[...] = (z - m) - jnp.log(jnp.sum(e, axis=1, keepdims=True))

    return pl.pallas_call(
        body,
        grid=(_N // bm,),
        in_specs=[
            pl.BlockSpec((bm, _OUT_C), lambda i: (i, 0)),
            pl.BlockSpec((bm, _OUT_C), lambda i: (i, 0)),
            pl.BlockSpec((1, _OUT_C), lambda i: (0, 0)),
        ],
        out_specs=pl.BlockSpec((bm, _OUT_C), lambda i: (i, 0)),
        out_shape=jax.ShapeDtypeStruct((_N, _OUT_C), jnp.float32),
    )(q0, q1, b_out.reshape(1, _OUT_C))


def kernel(x, adj, edge_weight, W_in, b_in, W_out, b_out):
    nw = 32
    ep = nw * _NCHUNK * _K
    pad = ep - _E
    src = jnp.concatenate([adj[0], jnp.zeros((pad,), jnp.int32)])
    dst = jnp.concatenate([adj[1], jnp.zeros((pad,), jnp.int32)])
    ew = jnp.concatenate([edge_weight, jnp.zeros((pad,), jnp.float32)])

    p1 = _make_spmm(_N, _IN_C, True)(x, src, dst, ew)
    h2 = _tc_mid(p1[0], p1[1], W_in, W_out, b_in)
    p2 = _make_spmm(_N, _OUT_C, False)(h2, src, dst, ew)
    return _tc_out(p2[0], p2[1], b_out)
